# Initial kernel scaffold; baseline (speedup 1.0000x reference)
import jax
import jax.numpy as jnp
from jax.experimental import pallas as pl


def kernel(inputs, epsilon):
    raise NotImplementedError("write your pallas kernel here")


# speedup vs baseline: 15.9996x; 15.9996x over previous
import functools
import jax
import jax.numpy as jnp
from jax import lax
from jax.experimental import pallas as pl
from jax.experimental.pallas import tpu as pltpu
from jax.experimental.pallas import tpu_sc as plsc

_B, _L, _M = 1024, 64, 256
_G = 8
_NG = _L // _G
_NCODE = 1 << _G
_LANES = 16
_BT = 32
_RPT = _BT * _NG


def _prep_body(inp_ref, eps_ref, idx_ref, table_ref):
    x = inp_ref[...]
    xg = x.reshape(_B, _NG, _G)
    w = lax.broadcasted_iota(jnp.int32, (1, 1, _G), 2)
    codes = jnp.sum(xg << w, axis=-1)
    g_iota = lax.broadcasted_iota(jnp.int32, (1, _NG), 1)
    rows = codes * _NG + g_iota
    idx_ref[...] = rows.reshape(2, _LANES, 2, 128)

    eps = eps_ref[...]
    E = jnp.transpose(eps, (0, 2, 1))
    for _ in range(3):
        c, ng, m = E.shape
        A = E.reshape(c, ng // 2, 2, m)
        lo = A[:, :, 0, :]
        hi = A[:, :, 1, :]
        E = (hi[:, None] * lo[None, :]).reshape(c * c, ng // 2, m)
    table_ref[...] = E.reshape(_NCODE * _NG, _M)


def _prep(inputs, epsilon):
    return pl.pallas_call(
        _prep_body,
        out_shape=(
            jax.ShapeDtypeStruct((2, _LANES, 2, 128), jnp.int32),
            jax.ShapeDtypeStruct((_NCODE * _NG, _M), jnp.float32),
        ),
    )(inputs, epsilon)


def _sc_call(table, idx):
    mesh = plsc.VectorSubcoreMesh(
        core_axis_name="c", subcore_axis_name="s", num_cores=2, num_subcores=16
    )

    @functools.partial(
        pl.kernel,
        mesh=mesh,
        out_type=jax.ShapeDtypeStruct((_B,), jnp.float32),
        scratch_types=[
            pltpu.VMEM((2, 128), jnp.int32),
            pltpu.VMEM((_RPT, _M), jnp.float32),
            pltpu.VMEM((_BT,), jnp.float32),
            pltpu.SemaphoreType.DMA,
            pltpu.SemaphoreType.DMA,
        ],
    )
    def sc_kernel(table_hbm, idx_hbm, out_hbm, idx_v, rows_v, out_v,
                  sem_i, sem_g):
        c = lax.axis_index("c")
        s = lax.axis_index("s")
        w = c * _LANES + s
        pltpu.async_copy(idx_hbm.at[c, s], idx_v, sem_i).wait()
        cp0 = pltpu.async_copy(
            table_hbm.at[idx_v.at[0]], rows_v.at[pl.ds(0, 128)], sem_g)
        cp1 = pltpu.async_copy(
            table_hbm.at[idx_v.at[1]], rows_v.at[pl.ds(128, 128)], sem_g)
        cp0.wait()
        cp1.wait()

        lane = lax.broadcasted_iota(jnp.int32, (_LANES,), 0)

        def vgroup_body(vg, carry):
            out_vec = jnp.zeros((_LANES,), jnp.float32)
            for k in range(_LANES):
                r0 = (vg * _LANES + k) * _NG
                tot = None
                for chunk in range(_M // _LANES):
                    sl = pl.ds(chunk * _LANES, _LANES)
                    acc = rows_v[r0, sl]
                    for g in range(1, _NG):
                        acc = acc * rows_v[r0 + g, sl]
                    tot = acc if tot is None else tot + acc
                for st in range(4):
                    perm = lane ^ (1 << st)
                    tot = tot + tot.at[perm].get(mode="promise_in_bounds")
                out_vec = jnp.where(lane == k, tot, out_vec)
            out_v[pl.ds(vg * _LANES, _LANES)] = out_vec
            return carry

        lax.fori_loop(0, _BT // _LANES, vgroup_body, 0)

        pltpu.sync_copy(out_v, out_hbm.at[pl.ds(w * _BT, _BT)])

    return sc_kernel(table, idx)


def kernel(inputs, epsilon):
    idx, table = _prep(inputs, epsilon)
    return _sc_call(table, idx)
